# Initial kernel scaffold; baseline (speedup 1.0000x reference)
import jax
import jax.numpy as jnp
from jax.experimental import pallas as pl


def kernel(x, edge_index, edge_attr, batch, Wrel0, brel0, Wroot0, Wrel1, brel1, Wroot1, Wrel2, brel2, Wroot2, Wrel3, brel3, Wroot3, Wrel4, brel4, Wroot4, Wm0, bm0, Wm1, bm1, Wm2, bm2):
    raise NotImplementedError("write your pallas kernel here")


# speedup vs baseline: 6.0978x; 6.0978x over previous
import functools

import jax
import jax.numpy as jnp
from jax import lax
from jax.experimental import pallas as pl
from jax.experimental.pallas import tpu as pltpu
from jax.experimental.pallas import tpu_sc as plsc

N = 10000
NPAD = 10240
E = 320000
G = 64

NC = 2
NS = 16
NW = NC * NS
EPW = E // NW
K = 80
NCH = EPW // K
RPT = NPAD // NS

BR = 1024
NB = NPAD // BR


def _make_sc_segsum(W: int):
    mesh = plsc.VectorSubcoreMesh(core_axis_name="c", subcore_axis_name="s")

    @functools.partial(
        pl.kernel,
        mesh=mesh,
        compiler_params=pltpu.CompilerParams(use_tc_tiling_on_sc=False),
        out_type=jax.ShapeDtypeStruct((NC, NPAD, W), jnp.float32),
        scratch_types=[
            pltpu.VMEM((NCH, K), jnp.int32),
            pltpu.VMEM((NCH, K), jnp.int32),
            pltpu.VMEM((NCH, K), jnp.float32),
            pltpu.VMEM((K, W), jnp.float32),
            pltpu.VMEM_SHARED((NPAD, W), jnp.float32),
            pltpu.SemaphoreType.DMA,
        ],
    )
    def seg_kernel(h_hbm, src_hbm, dst_hbm, ew_hbm, out_hbm,
                   src_v, dst_v, ew_v, rows_v, acc, sem):
        c = lax.axis_index("c")
        s = lax.axis_index("s")
        wid = s * NC + c

        pltpu.sync_copy(src_hbm.at[wid], src_v)
        pltpu.sync_copy(dst_hbm.at[wid], dst_v)
        pltpu.sync_copy(ew_hbm.at[wid], ew_v)

        def zrow(r, _):
            for w in range(W // 16):
                rows_v[r, pl.ds(w * 16, 16)] = jnp.zeros((16,), jnp.float32)
            return 0
        lax.fori_loop(0, K, zrow, 0)
        for j in range(RPT // K):
            pltpu.sync_copy(rows_v, acc.at[pl.ds(s * RPT + j * K, K)])
        plsc.subcore_barrier()

        def chunk(ci, _):
            pltpu.async_copy(h_hbm.at[src_v.at[ci]], rows_v, sem).wait()

            def scale(q, _):
                ew16 = ew_v[ci, pl.ds(q * 16, 16)]
                for j in range(16):
                    sval = ew16[j]
                    e = q * 16 + j
                    for w in range(W // 16):
                        rows_v[e, pl.ds(w * 16, 16)] = (
                            rows_v[e, pl.ds(w * 16, 16)] * sval)
                return 0
            lax.fori_loop(0, K // 16, scale, 0)
            pltpu.sync_copy(rows_v, acc.at[dst_v.at[ci]], add=True)
            return 0
        lax.fori_loop(0, NCH, chunk, 0)
        plsc.subcore_barrier()

        pltpu.sync_copy(acc.at[pl.ds(s * RPT, RPT)],
                        out_hbm.at[c, pl.ds(s * RPT, RPT)])

    return seg_kernel


_SC_SEGSUM = {W: _make_sc_segsum(W) for W in (16, 64, 128)}


def _make_tc_layer(Wa, din, dout, apply_wr, wnext_dim=None):
    def body(*refs):
        if apply_wr and wnext_dim is not None:
            agg_ref, h_ref, wr_ref, wo_ref, b_ref, wy_ref, hout_ref, yout_ref = refs
        elif apply_wr:
            agg_ref, h_ref, wr_ref, wo_ref, b_ref, hout_ref = refs
        elif wnext_dim is not None:
            agg_ref, h_ref, wo_ref, b_ref, wy_ref, hout_ref, yout_ref = refs
        else:
            agg_ref, h_ref, wo_ref, b_ref, hout_ref = refs
        aggs = agg_ref[0] + agg_ref[1]
        if apply_wr:
            t = jnp.dot(aggs, wr_ref[...], preferred_element_type=jnp.float32)
        else:
            t = aggs
        hnew = t + jnp.dot(h_ref[...], wo_ref[...],
                           preferred_element_type=jnp.float32) + b_ref[...]
        hnew = jnp.maximum(hnew, 0.0)
        hout_ref[...] = hnew
        if wnext_dim is not None:
            yout_ref[...] = jnp.dot(hnew, wy_ref[...],
                                    preferred_element_type=jnp.float32)

    in_specs = [
        pl.BlockSpec((NC, BR, Wa), lambda i: (0, i, 0)),
        pl.BlockSpec((BR, din), lambda i: (i, 0)),
    ]
    if apply_wr:
        in_specs.append(pl.BlockSpec((Wa, dout), lambda i: (0, 0)))
    in_specs.append(pl.BlockSpec((din, dout), lambda i: (0, 0)))
    in_specs.append(pl.BlockSpec((1, dout), lambda i: (0, 0)))
    out_specs = [pl.BlockSpec((BR, dout), lambda i: (i, 0))]
    out_shape = [jax.ShapeDtypeStruct((NPAD, dout), jnp.float32)]
    if wnext_dim is not None:
        in_specs.append(pl.BlockSpec((dout, wnext_dim), lambda i: (0, 0)))
        out_specs.append(pl.BlockSpec((BR, wnext_dim), lambda i: (i, 0)))
        out_shape.append(jax.ShapeDtypeStruct((NPAD, wnext_dim), jnp.float32))

    f = pl.pallas_call(
        body,
        grid=(NB,),
        in_specs=in_specs,
        out_specs=out_specs if len(out_specs) > 1 else out_specs[0],
        out_shape=out_shape if len(out_shape) > 1 else out_shape[0],
    )
    return f


def _make_tc_pool(Wa, din, dout):
    def body(agg_ref, h_ref, wo_ref, b_ref, batch_ref, sums_ref, cnt_ref):
        i = pl.program_id(0)
        hnew = agg_ref[0] + agg_ref[1] + jnp.dot(
            h_ref[...], wo_ref[...], preferred_element_type=jnp.float32)
        hnew = jnp.maximum(hnew + b_ref[...], 0.0)
        bvec = batch_ref[0, 0]
        oh = (bvec[:, None] == lax.broadcasted_iota(jnp.int32, (1, G), 1)
              ).astype(jnp.float32)
        bs = lax.dot_general(oh, hnew, (((0,), (0,)), ((), ())),
                             preferred_element_type=jnp.float32)
        bc = lax.dot_general(oh, jnp.ones((BR, dout), jnp.float32),
                             (((0,), (0,)), ((), ())),
                             preferred_element_type=jnp.float32)

        @pl.when(i == 0)
        def _():
            sums_ref[...] = bs
            cnt_ref[...] = bc

        @pl.when(i > 0)
        def _():
            sums_ref[...] += bs
            cnt_ref[...] += bc

    return pl.pallas_call(
        body,
        grid=(NB,),
        in_specs=[
            pl.BlockSpec((NC, BR, Wa), lambda i: (0, i, 0)),
            pl.BlockSpec((BR, din), lambda i: (i, 0)),
            pl.BlockSpec((din, dout), lambda i: (0, 0)),
            pl.BlockSpec((1, dout), lambda i: (0, 0)),
            pl.BlockSpec((1, 1, BR), lambda i: (i, 0, 0)),
        ],
        out_specs=[
            pl.BlockSpec((G, dout), lambda i: (0, 0)),
            pl.BlockSpec((G, dout), lambda i: (0, 0)),
        ],
        out_shape=[
            jax.ShapeDtypeStruct((G, dout), jnp.float32),
            jax.ShapeDtypeStruct((G, dout), jnp.float32),
        ],
    )


def _make_tc_mlp():
    def body(sums_ref, cnt_ref, w0, b0, w1, b1, w2, b2, out_ref):
        h = sums_ref[...] / jnp.maximum(cnt_ref[...], 1.0)
        h = jnp.maximum(jnp.dot(h, w0[...], preferred_element_type=jnp.float32)
                        + b0[...], 0.0)
        h = jnp.maximum(jnp.dot(h, w1[...], preferred_element_type=jnp.float32)
                        + b1[...], 0.0)
        out_ref[...] = jnp.dot(h, w2[...],
                               preferred_element_type=jnp.float32) + b2[...]

    return pl.pallas_call(
        body,
        out_shape=jax.ShapeDtypeStruct((G, 1), jnp.float32),
    )


_TC0 = _make_tc_layer(16, 16, 64, apply_wr=True)
_TC1 = _make_tc_layer(64, 64, 128, apply_wr=True)
_TC2 = _make_tc_layer(128, 128, 256, apply_wr=True, wnext_dim=128)
_TC3 = _make_tc_layer(128, 256, 128, apply_wr=False, wnext_dim=64)
_TC4 = _make_tc_pool(64, 128, 64)
_TCMLP = _make_tc_mlp()


def kernel(x, edge_index, edge_attr, batch,
           Wrel0, brel0, Wroot0,
           Wrel1, brel1, Wroot1,
           Wrel2, brel2, Wroot2,
           Wrel3, brel3, Wroot3,
           Wrel4, brel4, Wroot4,
           Wm0, bm0, Wm1, bm1, Wm2, bm2):
    src3 = edge_index[0].reshape(NW, NCH, K)
    dst3 = edge_index[1].reshape(NW, NCH, K)
    ew3 = edge_attr.reshape(NW, NCH, K)

    xpad = jnp.zeros((NPAD, 16), jnp.float32).at[:N, :5].set(x)
    batch_pad = jnp.full((NPAD,), G, jnp.int32).at[:N].set(batch)
    batch3 = batch_pad.reshape(NB, 1, BR)

    Wr0p = jnp.zeros((16, 64), jnp.float32).at[:5].set(Wrel0)
    Wo0p = jnp.zeros((16, 64), jnp.float32).at[:5].set(Wroot0)

    b0 = brel0.reshape(1, -1)
    b1 = brel1.reshape(1, -1)
    b2 = brel2.reshape(1, -1)
    b3 = brel3.reshape(1, -1)
    b4 = brel4.reshape(1, -1)

    agg0 = _SC_SEGSUM[16](xpad, src3, dst3, ew3)
    h1 = _TC0(agg0, xpad, Wr0p, Wo0p, b0)
    agg1 = _SC_SEGSUM[64](h1, src3, dst3, ew3)
    h2 = _TC1(agg1, h1, Wrel1, Wroot1, b1)
    agg2 = _SC_SEGSUM[128](h2, src3, dst3, ew3)
    h3, y3 = _TC2(agg2, h2, Wrel2, Wroot2, b2, Wrel3)
    agg3 = _SC_SEGSUM[128](y3, src3, dst3, ew3)
    h4, y4 = _TC3(agg3, h3, Wroot3, b3, Wrel4)
    agg4 = _SC_SEGSUM[64](y4, src3, dst3, ew3)
    sums, cnt = _TC4(agg4, h4, Wroot4, b4, batch3)
    out = _TCMLP(sums, cnt, Wm0, bm0.reshape(1, -1),
                 Wm1, bm1.reshape(1, -1), Wm2, bm2.reshape(1, -1))
    return out
